# Initial kernel scaffold; baseline (speedup 1.0000x reference)
import jax
import jax.numpy as jnp
from jax.experimental import pallas as pl


def kernel(dense_x, discrete_x, dense_W, dense_b, w1_tables, emb_tables):
    raise NotImplementedError("write your pallas kernel here")


# speedup vs baseline: 1.2301x; 1.2301x over previous
import jax
import jax.numpy as jnp
from jax import lax
from jax.experimental import pallas as pl
from jax.experimental.pallas import tpu as pltpu
from jax.experimental.pallas import tpu_sc as plsc

_B = 16384
_F = 26
_V = 100000
_D = 16
_DENSE = 13

_NC = 2
_NS = 16
_NW = _NC * _NS
_RPW = _B // _NW
_CB = 64
_NCHUNK = _RPW // _CB
_G = _F * _CB
_NIDX = _G // 128


def _fm_body(emb_hbm, w1_hbm, idx_hbm, dx_hbm, wb_hbm, out_hbm,
             idx_vm, dx_vm, fidx_vm, emb_vm, w1_vm, wb_vm, out_vm, sem):
    wid = lax.axis_index("s") * _NC + lax.axis_index("c")
    lane = lax.iota(jnp.int32, 16)

    pltpu.sync_copy(wb_hbm, wb_vm)
    wv = wb_vm[...]
    wsplat = [jnp.sum(jnp.where(lane == k, wv, 0.0))
              for k in range(_DENSE + 1)]

    def chunk(c, carry):
        base = wid * _RPW + c * _CB
        cps_in = []
        for f in range(_F):
            cps_in.append(pltpu.async_copy(
                idx_hbm.at[pl.ds(f * _B + base, _CB)], idx_vm.at[f], sem))
        for k in range(_DENSE):
            cps_in.append(pltpu.async_copy(
                dx_hbm.at[pl.ds(k * _B + base, _CB)], dx_vm.at[k], sem))
        for cp in cps_in:
            cp.wait()

        for f in range(_F):
            for g in range(_CB // 16):
                v = idx_vm[f, pl.ds(g * 16, 16)] + f * _V
                p = f * _CB + g * 16
                fidx_vm[p // 128, pl.ds(p % 128, 16)] = v

        copies = []
        for i in range(_NIDX):
            copies.append(pltpu.async_copy(
                emb_hbm.at[fidx_vm.at[i]], emb_vm.at[pl.ds(i * 128, 128)], sem))
            copies.append(pltpu.async_copy(
                w1_hbm.at[fidx_vm.at[i]], w1_vm.at[i], sem))
        for cp in copies:
            cp.wait()

        for g in range(_CB // 16):
            dacc = wsplat[_DENSE] + jnp.zeros((16,), jnp.float32)
            for k in range(_DENSE):
                dacc = dacc + dx_vm[k, pl.ds(g * 16, 16)] * wsplat[k]
            w1acc = dacc
            for f in range(_F):
                p = f * _CB + g * 16
                w1acc = w1acc + w1_vm[p // 128, pl.ds(p % 128, 16)]
            eres = w1acc
            for j in range(16):
                jj = g * 16 + j
                acc_s = emb_vm[jj, :]
                acc_q = acc_s * acc_s
                for f in range(1, _F):
                    e = emb_vm[f * _CB + jj, :]
                    acc_s = acc_s + e
                    acc_q = acc_q + e * e
                r = 0.5 * jnp.sum(acc_s * acc_s - acc_q)
                eres = jnp.where(lane == j, eres + r, eres)
            out_vm[pl.ds(g * 16, 16)] = eres
        pltpu.sync_copy(out_vm, out_hbm.at[pl.ds(base, _CB)])
        return carry

    lax.fori_loop(0, _NCHUNK, chunk, 0)


_fm_call = pl.kernel(
    _fm_body,
    out_type=jax.ShapeDtypeStruct((_B,), jnp.float32),
    mesh=plsc.VectorSubcoreMesh(core_axis_name="c", subcore_axis_name="s"),
    compiler_params=pltpu.CompilerParams(
        needs_layout_passes=False, use_tc_tiling_on_sc=False),
    scratch_types=[
        pltpu.VMEM((_F, _CB), jnp.int32),
        pltpu.VMEM((_DENSE, _CB), jnp.float32),
        pltpu.VMEM((_NIDX, 128), jnp.int32),
        pltpu.VMEM((_G, _D), jnp.float32),
        pltpu.VMEM((_NIDX, 128), jnp.float32),
        pltpu.VMEM((16,), jnp.float32),
        pltpu.VMEM((_CB,), jnp.float32),
        pltpu.SemaphoreType.DMA,
    ],
)


@jax.jit
def kernel(dense_x, discrete_x, dense_W, dense_b, w1_tables, emb_tables):
    idx_t = discrete_x.astype(jnp.int32).T.reshape(_F * _B)
    dx_t = dense_x.T.reshape(_DENSE * _B)
    wb = jnp.concatenate([dense_W[:, 0], dense_b,
                          jnp.zeros((2,), jnp.float32)])
    emb_flat = emb_tables.reshape(_F * _V, _D)
    w1_flat = w1_tables.reshape(_F * _V)
    out = _fm_call(emb_flat, w1_flat, idx_t, dx_t, wb)
    return out[:, None]
